# baseline (speedup 1.0000x reference)
import dataclasses
import functools

import jax
import jax.numpy as jnp
from jax import lax
from jax.experimental import pallas as pl
from jax.experimental.pallas import tpu as pltpu
from jax.experimental.pallas import tpu_sc as plsc

_B, _C, _H, _W = 4, 96, 384, 384
_NPIX = _B * _H * _W
_NC, _NS, _LANES = 2, 16, 16
_NW = _NC * _NS
_PPW = _NPIX // _NW
_GP = 32
_NWIN = _PPW // _GP


def _prep_body(flow_ref, idxq_ref, wts_ref):
    b = pl.program_id(0)
    fx = flow_ref[0, 0]
    fy = flow_ref[0, 1]
    xw = lax.broadcasted_iota(jnp.int32, (_H, _W), 1).astype(jnp.float32)
    yh = lax.broadcasted_iota(jnp.int32, (_H, _W), 0).astype(jnp.float32)
    x = jnp.clip(xw + fx, 0.0, float(_W - 1))
    y = jnp.clip(yh + fy, 0.0, float(_H - 1))
    x0 = jnp.minimum(jnp.floor(x), float(_W - 2))
    y0 = jnp.minimum(jnp.floor(y), float(_H - 2))
    wx1 = x - x0
    wx0 = 1.0 - wx1
    wy1 = y - y0
    wy0 = 1.0 - wy1
    x0i = x0.astype(jnp.int32)
    y0i = y0.astype(jnp.int32)
    q0 = (b * _H + y0i) * _W + x0i
    idxq_ref[0, 0] = q0
    idxq_ref[0, 1] = q0 + 1
    idxq_ref[0, 2] = q0 + _W
    idxq_ref[0, 3] = q0 + _W + 1
    wts_ref[0, 0] = wy0 * wx0
    wts_ref[0, 1] = wy0 * wx1
    wts_ref[0, 2] = wy1 * wx0
    wts_ref[0, 3] = wy1 * wx1


def _prep(flow):
    return pl.pallas_call(
        _prep_body,
        grid=(_B,),
        in_specs=[pl.BlockSpec((1, 2, _H, _W), lambda b: (b, 0, 0, 0))],
        out_specs=[
            pl.BlockSpec((1, 4, _H, _W), lambda b: (b, 0, 0, 0)),
            pl.BlockSpec((1, 4, _H, _W), lambda b: (b, 0, 0, 0)),
        ],
        out_shape=[
            jax.ShapeDtypeStruct((_B, 4, _H, _W), jnp.int32),
            jax.ShapeDtypeStruct((_B, 4, _H, _W), jnp.float32),
        ],
    )(flow)


def _sc_warp(img_rows, idx_flat, wts):
    mesh = plsc.VectorSubcoreMesh(core_axis_name="c", subcore_axis_name="s")
    cp = pltpu.CompilerParams()
    for f, v in (("needs_layout_passes", False), ("use_tc_tiling_on_sc", False)):
        if f in pltpu.CompilerParams.__dataclass_fields__:
            cp = dataclasses.replace(cp, **{f: v})

    @functools.partial(
        pl.kernel,
        mesh=mesh,
        compiler_params=cp,
        out_type=jax.ShapeDtypeStruct((_NPIX, _C), jnp.float32),
        scratch_types=[
            pltpu.VMEM((4 * _GP,), jnp.int32),
            pltpu.VMEM((4 * _GP,), jnp.float32),
            pltpu.VMEM((4 * _GP, _C), jnp.float32),
            pltpu.VMEM((_GP, _C), jnp.float32),
            pltpu.SemaphoreType.DMA,
        ],
    )
    def warp_kernel(img_hbm, idx_hbm, wts_hbm, out_hbm, idx_v, w_v, r_v, o_v, sem):
        wid = lax.axis_index("s") * _NC + lax.axis_index("c")
        base = wid * _PPW

        @pl.loop(0, _NWIN)
        def _win(n):
            p0 = base + n * _GP
            pltpu.sync_copy(idx_hbm.at[pl.ds(4 * p0, 4 * _GP)], idx_v)
            pltpu.sync_copy(wts_hbm.at[pl.ds(4 * p0, 4 * _GP)], w_v)
            pltpu.async_copy(img_hbm.at[idx_v], r_v, sem).wait()

            @pl.loop(0, _GP)
            def _px(g):
                b4 = 4 * g
                w0 = plsc.load_gather(w_v, [jnp.full((_LANES,), b4, jnp.int32)])
                w1 = plsc.load_gather(w_v, [jnp.full((_LANES,), b4 + 1, jnp.int32)])
                w2 = plsc.load_gather(w_v, [jnp.full((_LANES,), b4 + 2, jnp.int32)])
                w3 = plsc.load_gather(w_v, [jnp.full((_LANES,), b4 + 3, jnp.int32)])
                for k in range(_C // _LANES):
                    s = pl.ds(k * _LANES, _LANES)
                    o_v[g, s] = (w0 * r_v[b4, s] + w1 * r_v[b4 + 1, s]
                                 + w2 * r_v[b4 + 2, s] + w3 * r_v[b4 + 3, s])

            pltpu.sync_copy(o_v, out_hbm.at[pl.ds(p0, _GP)])

    return warp_kernel(img_rows, idx_flat, wts)


def kernel(img, flow):
    idxq, wts = _prep(flow)
    hw = _H * _W
    idx_flat = idxq.reshape(_B, 4, hw).transpose(0, 2, 1).reshape(4 * _NPIX)
    wts_flat = wts.reshape(_B, 4, hw).transpose(0, 2, 1).reshape(4 * _NPIX)
    img_rows = img.transpose(0, 2, 3, 1).reshape(_NPIX, _C)
    out_rows = _sc_warp(img_rows, idx_flat, wts_flat)
    return out_rows.reshape(_B, _H, _W, _C).transpose(0, 3, 1, 2)


# speedup vs baseline: 1.4888x; 1.4888x over previous
import dataclasses
import functools

import jax
import jax.numpy as jnp
from jax import lax
from jax.experimental import pallas as pl
from jax.experimental.pallas import tpu as pltpu
from jax.experimental.pallas import tpu_sc as plsc

_B, _C, _H, _W = 4, 96, 384, 384
_NPIX = _B * _H * _W
_NC, _NS, _LANES = 2, 16, 16
_NW = _NC * _NS
_PPW = _NPIX // _NW
_GP = 32
_NWIN = _PPW // _GP
_NBUF = 3


def _prep_body(flow_ref, idxq_ref, wts_ref):
    b = pl.program_id(0)
    fx = flow_ref[0, 0]
    fy = flow_ref[0, 1]
    xw = lax.broadcasted_iota(jnp.int32, (_H, _W), 1).astype(jnp.float32)
    yh = lax.broadcasted_iota(jnp.int32, (_H, _W), 0).astype(jnp.float32)
    x = jnp.clip(xw + fx, 0.0, float(_W - 1))
    y = jnp.clip(yh + fy, 0.0, float(_H - 1))
    x0 = jnp.minimum(jnp.floor(x), float(_W - 2))
    y0 = jnp.minimum(jnp.floor(y), float(_H - 2))
    wx1 = x - x0
    wx0 = 1.0 - wx1
    wy1 = y - y0
    wy0 = 1.0 - wy1
    x0i = x0.astype(jnp.int32)
    y0i = y0.astype(jnp.int32)
    q0 = (b * _H + y0i) * _W + x0i
    idxq_ref[0, 0] = q0
    idxq_ref[0, 1] = q0 + 1
    idxq_ref[0, 2] = q0 + _W
    idxq_ref[0, 3] = q0 + _W + 1
    wts_ref[0, 0] = wy0 * wx0
    wts_ref[0, 1] = wy0 * wx1
    wts_ref[0, 2] = wy1 * wx0
    wts_ref[0, 3] = wy1 * wx1


def _prep(flow):
    return pl.pallas_call(
        _prep_body,
        grid=(_B,),
        in_specs=[pl.BlockSpec((1, 2, _H, _W), lambda b: (b, 0, 0, 0))],
        out_specs=[
            pl.BlockSpec((1, 4, _H, _W), lambda b: (b, 0, 0, 0)),
            pl.BlockSpec((1, 4, _H, _W), lambda b: (b, 0, 0, 0)),
        ],
        out_shape=[
            jax.ShapeDtypeStruct((_B, 4, _H, _W), jnp.int32),
            jax.ShapeDtypeStruct((_B, 4, _H, _W), jnp.float32),
        ],
    )(flow)


def _sc_warp(img_rows, idx_flat, wts):
    mesh = plsc.VectorSubcoreMesh(core_axis_name="c", subcore_axis_name="s")
    cp = pltpu.CompilerParams()
    for f, v in (("needs_layout_passes", False), ("use_tc_tiling_on_sc", False)):
        if f in pltpu.CompilerParams.__dataclass_fields__:
            cp = dataclasses.replace(cp, **{f: v})

    nbuf = _NBUF

    @functools.partial(
        pl.kernel,
        mesh=mesh,
        compiler_params=cp,
        out_type=jax.ShapeDtypeStruct((_NPIX, _C), jnp.float32),
        scratch_types=[
            pltpu.VMEM((nbuf, 4 * _GP), jnp.int32),
            pltpu.VMEM((nbuf, 4 * _GP), jnp.float32),
            pltpu.VMEM((nbuf, 4 * _GP, _C), jnp.float32),
            pltpu.VMEM((nbuf, _GP, _C), jnp.float32),
            pltpu.SemaphoreType.DMA((nbuf,)),
            pltpu.SemaphoreType.DMA((nbuf,)),
            pltpu.SemaphoreType.DMA((nbuf,)),
        ],
    )
    def warp_kernel(img_hbm, idx_hbm, wts_hbm, out_hbm,
                    idx_v, w_v, r_v, o_v, sem_ld, sem_g, sem_st):
        wid = lax.axis_index("s") * _NC + lax.axis_index("c")
        base = wid * _PPW

        def issue_load(win, j):
            p4 = 4 * (base + win * _GP)
            pltpu.async_copy(idx_hbm.at[pl.ds(p4, 4 * _GP)], idx_v.at[j],
                             sem_ld.at[j])
            pltpu.async_copy(wts_hbm.at[pl.ds(p4, 4 * _GP)], w_v.at[j],
                             sem_ld.at[j])

        def wait_load(win, j):
            p4 = 4 * (base + win * _GP)
            pltpu.make_async_copy(idx_hbm.at[pl.ds(p4, 4 * _GP)], idx_v.at[j],
                                  sem_ld.at[j]).wait()
            pltpu.make_async_copy(wts_hbm.at[pl.ds(p4, 4 * _GP)], w_v.at[j],
                                  sem_ld.at[j]).wait()

        def issue_gather(j):
            pltpu.async_copy(img_hbm.at[idx_v.at[j]], r_v.at[j], sem_g.at[j])

        def wait_gather(j):
            pltpu.make_async_copy(img_hbm.at[idx_v.at[j]], r_v.at[j],
                                  sem_g.at[j]).wait()

        def issue_store(win, j):
            pltpu.async_copy(o_v.at[j], out_hbm.at[pl.ds(base + win * _GP, _GP)],
                             sem_st.at[j])

        def wait_store(win, j):
            pltpu.make_async_copy(o_v.at[j], out_hbm.at[pl.ds(base + win * _GP, _GP)],
                                  sem_st.at[j]).wait()

        def combine(j):
            @pl.loop(0, _GP)
            def _px(g):
                b4 = 4 * g
                w0 = plsc.load_gather(w_v.at[j], [jnp.full((_LANES,), b4, jnp.int32)])
                w1 = plsc.load_gather(w_v.at[j], [jnp.full((_LANES,), b4 + 1, jnp.int32)])
                w2 = plsc.load_gather(w_v.at[j], [jnp.full((_LANES,), b4 + 2, jnp.int32)])
                w3 = plsc.load_gather(w_v.at[j], [jnp.full((_LANES,), b4 + 3, jnp.int32)])
                for k in range(_C // _LANES):
                    s = pl.ds(k * _LANES, _LANES)
                    o_v[j, g, s] = (w0 * r_v[j, b4, s] + w1 * r_v[j, b4 + 1, s]
                                    + w2 * r_v[j, b4 + 2, s] + w3 * r_v[j, b4 + 3, s])

        issue_load(0, 0)
        wait_load(0, 0)
        issue_gather(0)
        issue_load(1, 1)

        @pl.loop(0, _NWIN // nbuf)
        def _outer(wo):
            for j in range(nbuf):
                w = wo * nbuf + j
                s1 = (j + 1) % nbuf
                s2 = (j + 2) % nbuf

                @pl.when(w + 1 < _NWIN)
                def _():
                    wait_load(w + 1, s1)
                    issue_gather(s1)

                @pl.when(w + 2 < _NWIN)
                def _():
                    issue_load(w + 2, s2)

                wait_gather(j)

                @pl.when(w >= nbuf)
                def _():
                    wait_store(w - nbuf, j)

                combine(j)
                issue_store(w, j)

        for j in range(nbuf):
            wait_store(_NWIN - nbuf + j, (_NWIN - nbuf + j) % nbuf)

    return warp_kernel(img_rows, idx_flat, wts)


def kernel(img, flow):
    idxq, wts = _prep(flow)
    hw = _H * _W
    idx_flat = idxq.reshape(_B, 4, hw).transpose(0, 2, 1).reshape(4 * _NPIX)
    wts_flat = wts.reshape(_B, 4, hw).transpose(0, 2, 1).reshape(4 * _NPIX)
    img_rows = img.transpose(0, 2, 3, 1).reshape(_NPIX, _C)
    out_rows = _sc_warp(img_rows, idx_flat, wts_flat)
    return out_rows.reshape(_B, _H, _W, _C).transpose(0, 3, 1, 2)
